# baseline (speedup 1.0000x reference)
import jax
import jax.numpy as jnp
from jax.experimental import pallas as pl
from jax.experimental.pallas import tpu as pltpu

_S = 8


def _pool_body(x_ref, o_ref):
    j = pl.program_id(1)

    @pl.when(j == 0)
    def _():
        o_ref[...] = jnp.zeros_like(o_ref)

    x = x_ref[...]
    o_ref[...] += jnp.sum(x, axis=1)[None]


def kernel(features, h, w):
    n, c = features.shape
    g = 128 // c if (c < 128 and 128 % c == 0) else 1
    lanes = c * g
    rows = n // g if n % g == 0 else 0
    if rows % (_S * 8) != 0:
        rows = 0
    if rows == 0:
        target = -(-n // (g * _S * 8)) * (g * _S * 8)
        features = jnp.pad(features, ((0, target - n), (0, 0)))
        n = target
        rows = n // g
    xr = features.reshape(_S, rows // _S, lanes)

    stripe = rows // _S
    k = 1
    for cand in range(40, 0, -1):
        if (stripe // 8) % cand == 0:
            k = cand
            break
    bn = stripe // k

    partials = pl.pallas_call(
        _pool_body,
        grid=(2, k),
        in_specs=[pl.BlockSpec((_S // 2, bn, lanes), lambda i, j: (i, j, 0))],
        out_specs=pl.BlockSpec((1, _S // 2, lanes), lambda i, j: (i, 0, 0)),
        out_shape=jax.ShapeDtypeStruct((2, _S // 2, lanes), jnp.float32),
        compiler_params=pltpu.CompilerParams(
            dimension_semantics=("parallel", "arbitrary"),
        ),
    )(xr)
    total = jnp.sum(partials, axis=(0, 1)).reshape(g, c).sum(axis=0)
    return total / (h * w)
